# baseline (speedup 1.0000x reference)
import functools

import jax
import jax.numpy as jnp
from jax import lax
from jax.experimental import pallas as pl
from jax.experimental.pallas import tpu as pltpu
from jax.experimental.pallas import tpu_sc as plsc


def _make_sc(batch, seq_len, d_model):
    info = plsc.get_sparse_core_info()
    nw = info.num_cores * info.num_subcores
    rows = seq_len // nw
    mesh = plsc.VectorSubcoreMesh(core_axis_name="c", subcore_axis_name="s")

    @functools.partial(
        pl.kernel,
        mesh=mesh,
        out_type=jax.ShapeDtypeStruct((batch, seq_len, d_model), jnp.float32),
        scratch_types=[pltpu.SemaphoreType.DMA],
    )
    def k(pe_hbm, out_hbm, sem):
        wid = lax.axis_index("s") * info.num_cores + lax.axis_index("c")
        base = wid * rows
        copies = [
            pltpu.async_copy(
                pe_hbm.at[pl.ds(base, rows)],
                out_hbm.at[b, pl.ds(base, rows)],
                sem,
            )
            for b in range(batch)
        ]
        for c in copies:
            c.wait()

    return k


def kernel(x, pe):
    batch, seq_len = x.shape
    d_model = pe.shape[1]
    return _make_sc(batch, seq_len, d_model)(pe)


# speedup vs baseline: 53.5775x; 53.5775x over previous
import functools

import jax
import jax.numpy as jnp
from jax import lax
from jax.experimental import pallas as pl
from jax.experimental.pallas import tpu as pltpu
from jax.experimental.pallas import tpu_sc as plsc

_CHUNK = 16


def _make_sc(batch, seq_len, d_model):
    info = plsc.get_sparse_core_info()
    nw = info.num_cores * info.num_subcores
    rows = seq_len // nw
    chunks = rows // _CHUNK
    mesh = plsc.VectorSubcoreMesh(core_axis_name="c", subcore_axis_name="s")

    @functools.partial(
        pl.kernel,
        mesh=mesh,
        out_type=jax.ShapeDtypeStruct((batch, seq_len, d_model), jnp.float32),
        scratch_types=[
            pltpu.VMEM((_CHUNK, d_model), jnp.float32),
            pltpu.VMEM((_CHUNK, d_model), jnp.float32),
            pltpu.SemaphoreType.DMA,
            pltpu.SemaphoreType.DMA,
            pltpu.SemaphoreType.DMA,
            pltpu.SemaphoreType.DMA,
        ],
    )
    def k(pe_hbm, out_hbm, buf0, buf1, isem0, isem1, osem0, osem1):
        wid = lax.axis_index("s") * info.num_cores + lax.axis_index("c")
        base = wid * rows
        bufs = (buf0, buf1)
        isems = (isem0, isem1)
        osems = (osem0, osem1)
        in_cp = [None] * chunks
        out_cp = [None] * chunks

        def start_in(g):
            in_cp[g] = pltpu.async_copy(
                pe_hbm.at[pl.ds(base + g * _CHUNK, _CHUNK)],
                bufs[g % 2],
                isems[g % 2],
            )

        def start_out(g):
            out_cp[g] = [
                pltpu.async_copy(
                    bufs[g % 2],
                    out_hbm.at[b, pl.ds(base + g * _CHUNK, _CHUNK)],
                    osems[g % 2],
                )
                for b in range(batch)
            ]

        start_in(0)
        for g in range(chunks):
            in_cp[g].wait()
            if g >= 1:
                for c in out_cp[g - 1]:
                    c.wait()
            if g + 1 < chunks:
                start_in(g + 1)
            start_out(g)
        for c in out_cp[chunks - 1]:
            c.wait()

    return k


def kernel(x, pe):
    batch, seq_len = x.shape
    d_model = pe.shape[1]
    return _make_sc(batch, seq_len, d_model)(pe)


# speedup vs baseline: 54.0315x; 1.0085x over previous
import functools

import jax
import jax.numpy as jnp
from jax import lax
from jax.experimental import pallas as pl
from jax.experimental.pallas import tpu as pltpu
from jax.experimental.pallas import tpu_sc as plsc

_CHUNK = 16
_NBUF = 3


def _make_sc(batch, seq_len, d_model):
    info = plsc.get_sparse_core_info()
    nw = info.num_cores * info.num_subcores
    rows = seq_len // nw
    chunks = rows // _CHUNK
    mesh = plsc.VectorSubcoreMesh(core_axis_name="c", subcore_axis_name="s")

    scratch = [pltpu.VMEM((_CHUNK, d_model), jnp.float32)] * _NBUF
    scratch += [pltpu.SemaphoreType.DMA] * (2 * _NBUF)

    @functools.partial(
        pl.kernel,
        mesh=mesh,
        out_type=jax.ShapeDtypeStruct((batch, seq_len, d_model), jnp.float32),
        scratch_types=scratch,
    )
    def k(pe_hbm, out_hbm, *refs):
        bufs = refs[:_NBUF]
        isems = refs[_NBUF:2 * _NBUF]
        osems = refs[2 * _NBUF:]
        wid = lax.axis_index("s") * info.num_cores + lax.axis_index("c")
        base = wid * rows
        in_cp = [None] * chunks
        out_cp = [None] * chunks

        def start_in(g):
            in_cp[g] = pltpu.async_copy(
                pe_hbm.at[pl.ds(base + g * _CHUNK, _CHUNK)],
                bufs[g % _NBUF],
                isems[g % _NBUF],
            )

        def start_out(g):
            out_cp[g] = [
                pltpu.async_copy(
                    bufs[g % _NBUF],
                    out_hbm.at[b, pl.ds(base + g * _CHUNK, _CHUNK)],
                    osems[g % _NBUF],
                )
                for b in range(batch)
            ]

        for g in range(min(_NBUF, chunks)):
            start_in(g)
        for g in range(chunks):
            in_cp[g].wait()
            if g >= _NBUF - 1:
                nxt = g + 1
                if nxt - _NBUF >= 0 and out_cp[nxt - _NBUF] is not None:
                    for c in out_cp[nxt - _NBUF]:
                        c.wait()
                    out_cp[nxt - _NBUF] = None
                if nxt < chunks:
                    start_in(nxt)
            start_out(g)
        for cs in out_cp:
            if cs is not None:
                for c in cs:
                    c.wait()

    return k


def kernel(x, pe):
    batch, seq_len = x.shape
    d_model = pe.shape[1]
    return _make_sc(batch, seq_len, d_model)(pe)
